# baseline (speedup 1.0000x reference)
import functools

import jax
import jax.numpy as jnp
from jax import lax
from jax.experimental import pallas as pl
from jax.experimental.pallas import tpu as pltpu
from jax.experimental.pallas import tpu_sc as plsc

VOCAB = 1000000
EMBED_DIM = 64
BATCH = 16384
HIST = 50

NC = 2
NS = 16
NW = NC * NS

B_PER_W = BATCH // NW
BB = BATCH // 128
BB_W = B_PER_W // 128
CHUNK = 256
BB_C = CHUNK // 128
NCHUNK = 2 * HIST

_mesh = plsc.VectorSubcoreMesh(core_axis_name="c", subcore_axis_name="s")


@functools.partial(
    pl.kernel,
    mesh=_mesh,
    compiler_params=pltpu.CompilerParams(
        use_tc_tiling_on_sc=False, needs_layout_passes=False
    ),
    out_type=jax.ShapeDtypeStruct((HIST, 8, BB, 8, 128), jnp.float32),
    scratch_types=[
        pltpu.VMEM((HIST, B_PER_W), jnp.int32),
        pltpu.VMEM((CHUNK, 2 * EMBED_DIM), jnp.float32),
        pltpu.VMEM((CHUNK, 2 * EMBED_DIM), jnp.float32),
        pltpu.VMEM((BB_C, 8, 8, 129), jnp.float32),
        pltpu.VMEM((BB_C, 8, 8, 129), jnp.float32),
        pltpu.SemaphoreType.DMA,
        pltpu.SemaphoreType.DMA,
        pltpu.SemaphoreType.DMA,
        pltpu.SemaphoreType.DMA,
    ],
)
def _gather_kernel(
    tok_hbm, table_hbm, out_hbm,
    idx_v, rows0, rows1, tr0, tr1,
    gsem0, gsem1, wsem0, wsem1,
):
    wid = lax.axis_index("s") * NC + lax.axis_index("c")
    base = wid * B_PER_W
    gsems = (gsem0, gsem1)
    wsems = (wsem0, wsem1)
    rowbufs = (rows0, rows1)
    trbufs = (tr0, tr1)

    pltpu.sync_copy(tok_hbm.at[:, pl.ds(base, B_PER_W)], idx_v)

    lane = lax.iota(jnp.int32, 16)
    lane_hi = lane // 8
    lane_lo = lane % 8

    def _idx_slice(k):
        return idx_v.at[k // 2, pl.ds((k % 2) * CHUNK, CHUNK)]

    def _start_gather(k, b):
        pltpu.async_copy(table_hbm.at[_idx_slice(k)], rowbufs[b], gsems[b])

    def _wait_gather(k, b):
        pltpu.make_async_copy(
            table_hbm.at[_idx_slice(k)], rowbufs[b], gsems[b]
        ).wait()

    def _transpose(b):
        rows = rowbufs[b]
        tr = trbufs[b]

        @plsc.parallel_loop(0, CHUNK // 16, unroll=4)
        def vbody(v):
            bblv = jnp.full((16,), v // 8, jnp.int32)
            for b16 in range(16):
                b_row = v * 16 + b16
                biv = jnp.full((16,), (v % 8) * 16 + b16, jnp.int32)
                for c0 in range(0, EMBED_DIM, 16):
                    vals = rows[b_row, pl.ds(c0, 16)]
                    plsc.store_scatter(
                        tr, [bblv, lane_hi + (c0 // 8), lane_lo, biv], vals
                    )

    def _wb_copies(k, b):
        h = k // 2
        half = k % 2
        for bbl in range(BB_C):
            yield (
                trbufs[b].at[bbl, :, :, pl.ds(0, 128)],
                out_hbm.at[h, :, wid * BB_W + half * BB_C + bbl],
            )

    def _start_writeback(k, b):
        for src, dst in _wb_copies(k, b):
            pltpu.async_copy(src, dst, wsems[b])

    def _wait_writeback(k, b):
        for src, dst in _wb_copies(k, b):
            pltpu.make_async_copy(src, dst, wsems[b]).wait()

    for b in range(2):
        _start_gather(b, b)

    def _body(i, carry):
        for b in range(2):
            k = 2 * i + b
            _wait_gather(k, b)

            @pl.when(k >= 2)
            def _():
                _wait_writeback(k - 2, b)

            _transpose(b)
            _start_writeback(k, b)

            @pl.when(k + 2 < NCHUNK)
            def _():
                _start_gather(k + 2, b)

        return carry

    lax.fori_loop(0, NCHUNK // 2, _body, 0)

    for b in range(2):
        _wait_writeback(NCHUNK - 2 + b, b)


def kernel(token_id, embed_weight):
    tbl = jnp.pad(embed_weight.T, ((0, EMBED_DIM), (0, 0))).T
    p5 = _gather_kernel(token_id.T, tbl)
    return p5.transpose(2, 4, 0, 1, 3).reshape(BATCH, HIST, EMBED_DIM)


# speedup vs baseline: 1.0044x; 1.0044x over previous
import functools

import jax
import jax.numpy as jnp
from jax import lax
from jax.experimental import pallas as pl
from jax.experimental.pallas import tpu as pltpu
from jax.experimental.pallas import tpu_sc as plsc

VOCAB = 1000000
EMBED_DIM = 64
BATCH = 16384
HIST = 50

NC = 2
NS = 16
NW = NC * NS

B_PER_W = BATCH // NW
BB = BATCH // 128
BB_W = B_PER_W // 128
CHUNK = 256
BB_C = CHUNK // 128
NCHUNK = 2 * HIST

_mesh = plsc.VectorSubcoreMesh(core_axis_name="c", subcore_axis_name="s")


@functools.partial(
    pl.kernel,
    mesh=_mesh,
    compiler_params=pltpu.CompilerParams(
        use_tc_tiling_on_sc=False, needs_layout_passes=False
    ),
    out_type=jax.ShapeDtypeStruct((HIST, 8, BB, 8, 128), jnp.float32),
    scratch_types=[
        pltpu.VMEM((HIST, B_PER_W), jnp.int32),
        pltpu.VMEM((CHUNK, 2 * EMBED_DIM), jnp.float32),
        pltpu.VMEM((CHUNK, 2 * EMBED_DIM), jnp.float32),
        pltpu.VMEM((BB_C, 8, 8, 129), jnp.float32),
        pltpu.VMEM((BB_C, 8, 8, 129), jnp.float32),
        pltpu.SemaphoreType.DMA,
        pltpu.SemaphoreType.DMA,
        pltpu.SemaphoreType.DMA,
        pltpu.SemaphoreType.DMA,
    ],
)
def _gather_kernel(
    tok_hbm, table_hbm, out_hbm,
    idx_v, rows0, rows1, tr0, tr1,
    gsem0, gsem1, wsem0, wsem1,
):
    wid = lax.axis_index("s") * NC + lax.axis_index("c")
    base = wid * B_PER_W
    gsems = (gsem0, gsem1)
    wsems = (wsem0, wsem1)
    rowbufs = (rows0, rows1)
    trbufs = (tr0, tr1)

    pltpu.sync_copy(tok_hbm.at[:, pl.ds(base, B_PER_W)], idx_v)

    lane = lax.iota(jnp.int32, 16)
    lane_hi = lane // 8
    lane_lo = lane % 8

    def _idx_slice(k):
        return idx_v.at[k // 2, pl.ds((k % 2) * CHUNK, CHUNK)]

    def _start_gather(k, b):
        pltpu.async_copy(table_hbm.at[_idx_slice(k)], rowbufs[b], gsems[b])

    def _wait_gather(k, b):
        pltpu.make_async_copy(
            table_hbm.at[_idx_slice(k)], rowbufs[b], gsems[b]
        ).wait()

    def _transpose(b):
        rows = rowbufs[b]
        tr = trbufs[b]

        @plsc.parallel_loop(0, CHUNK // 16, unroll=4)
        def vbody(v):
            bblv = jnp.full((16,), v // 8, jnp.int32)
            for b16 in range(16):
                b_row = v * 16 + b16
                biv = jnp.full((16,), (v % 8) * 16 + b16, jnp.int32)
                for c0 in range(0, EMBED_DIM, 16):
                    vals = rows[b_row, pl.ds(c0, 16)]
                    plsc.store_scatter(
                        tr, [bblv, lane_hi + (c0 // 8), lane_lo, biv], vals
                    )

    def _wb_copies(k, b):
        h = k // 2
        half = k % 2
        for bbl in range(BB_C):
            yield (
                trbufs[b].at[bbl, :, :, pl.ds(0, 128)],
                out_hbm.at[h, :, wid * BB_W + half * BB_C + bbl],
            )

    def _start_writeback(k, b):
        for src, dst in _wb_copies(k, b):
            pltpu.async_copy(src, dst, wsems[b])

    def _wait_writeback(k, b):
        for src, dst in _wb_copies(k, b):
            pltpu.make_async_copy(src, dst, wsems[b]).wait()

    for b in range(2):
        _start_gather(b, b)

    def _body(i, carry):
        for b in range(2):
            k = 2 * i + b
            _wait_gather(k, b)

            @pl.when(k >= 2)
            def _():
                _wait_writeback(k - 2, b)

            _transpose(b)
            _start_writeback(k, b)

            @pl.when(k + 2 < NCHUNK)
            def _():
                _start_gather(k + 2, b)

        return carry

    lax.fori_loop(0, NCHUNK // 2, _body, 0)

    for b in range(2):
        _wait_writeback(NCHUNK - 2 + b, b)


def kernel(token_id, embed_weight):
    tbl = jnp.pad(embed_weight, ((0, 0), (0, EMBED_DIM)))
    p5 = _gather_kernel(token_id.T, tbl)
    return p5.transpose(2, 4, 0, 1, 3).reshape(BATCH, HIST, EMBED_DIM)


# speedup vs baseline: 1.0577x; 1.0531x over previous
import functools

import jax
import jax.numpy as jnp
from jax import lax
from jax.experimental import pallas as pl
from jax.experimental.pallas import tpu as pltpu
from jax.experimental.pallas import tpu_sc as plsc

VOCAB = 1000000
EMBED_DIM = 64
BATCH = 16384
HIST = 50

NC = 2
NS = 16
NW = NC * NS

B_PER_W = BATCH // NW
BB = BATCH // 128
BB_W = B_PER_W // 128
CHUNK = 256
BB_C = CHUNK // 128
NCHUNK = 2 * HIST

_mesh = plsc.VectorSubcoreMesh(core_axis_name="c", subcore_axis_name="s")


@functools.partial(
    pl.kernel,
    mesh=_mesh,
    compiler_params=pltpu.CompilerParams(
        use_tc_tiling_on_sc=False, needs_layout_passes=False
    ),
    out_type=jax.ShapeDtypeStruct((HIST, 8, BB, 8, 128), jnp.float32),
    scratch_types=[
        pltpu.VMEM((HIST, B_PER_W), jnp.int32),
        pltpu.VMEM((CHUNK, 2 * EMBED_DIM), jnp.float32),
        pltpu.VMEM((CHUNK, 2 * EMBED_DIM), jnp.float32),
        pltpu.VMEM((BB_C, 8, 8, 129), jnp.float32),
        pltpu.VMEM((BB_C, 8, 8, 129), jnp.float32),
        pltpu.SemaphoreType.DMA,
        pltpu.SemaphoreType.DMA,
        pltpu.SemaphoreType.DMA,
        pltpu.SemaphoreType.DMA,
    ],
)
def _gather_kernel(
    tok_hbm, table_hbm, out_hbm,
    idx_v, rows0, rows1, tr0, tr1,
    gsem0, gsem1, wsem0, wsem1,
):
    wid = lax.axis_index("s") * NC + lax.axis_index("c")
    base = wid * B_PER_W
    gsems = (gsem0, gsem1)
    wsems = (wsem0, wsem1)
    rowbufs = (rows0, rows1)
    trbufs = (tr0, tr1)

    pltpu.sync_copy(tok_hbm.at[:, pl.ds(base, B_PER_W)], idx_v)

    lane = lax.iota(jnp.int32, 16)
    lane_hi = lane // 8
    lane_lo = lane % 8

    def _idx_slice(k):
        return idx_v.at[k // 2, pl.ds((k % 2) * CHUNK, CHUNK)]

    def _start_gather(k, b):
        pltpu.async_copy(table_hbm.at[_idx_slice(k)], rowbufs[b], gsems[b])

    def _wait_gather(k, b):
        pltpu.make_async_copy(
            table_hbm.at[_idx_slice(k)], rowbufs[b], gsems[b]
        ).wait()

    def _transpose(b):
        rows = rowbufs[b]
        tr = trbufs[b]

        @plsc.parallel_loop(0, CHUNK // 16, unroll=2)
        def vbody(v):
            bblv = jnp.full((16,), v // 8, jnp.int32)
            for b16 in range(16):
                b_row = v * 16 + b16
                biv = jnp.full((16,), (v % 8) * 16 + b16, jnp.int32)
                for c0 in range(0, EMBED_DIM, 16):
                    vals = rows[b_row, pl.ds(c0, 16)]
                    plsc.store_scatter(
                        tr, [bblv, lane_hi + (c0 // 8), lane_lo, biv], vals
                    )

    def _wb_copies(k, b):
        h = k // 2
        half = k % 2
        for bbl in range(BB_C):
            yield (
                trbufs[b].at[bbl, :, :, pl.ds(0, 128)],
                out_hbm.at[h, :, wid * BB_W + half * BB_C + bbl],
            )

    def _start_writeback(k, b):
        for src, dst in _wb_copies(k, b):
            pltpu.async_copy(src, dst, wsems[b])

    def _wait_writeback(k, b):
        for src, dst in _wb_copies(k, b):
            pltpu.make_async_copy(src, dst, wsems[b]).wait()

    for b in range(2):
        _start_gather(b, b)

    def _body(i, carry):
        for b in range(2):
            k = 2 * i + b
            _wait_gather(k, b)

            @pl.when(k >= 2)
            def _():
                _wait_writeback(k - 2, b)

            _transpose(b)
            _start_writeback(k, b)

            @pl.when(k + 2 < NCHUNK)
            def _():
                _start_gather(k + 2, b)

        return carry

    lax.fori_loop(0, NCHUNK // 2, _body, 0)

    for b in range(2):
        _wait_writeback(NCHUNK - 2 + b, b)


def kernel(token_id, embed_weight):
    tbl = jnp.pad(embed_weight, ((0, 0), (0, EMBED_DIM)))
    p5 = _gather_kernel(token_id.T, tbl)
    return p5.transpose(2, 4, 0, 1, 3).reshape(BATCH, HIST, EMBED_DIM)


# speedup vs baseline: 1.0733x; 1.0147x over previous
import functools

import jax
import jax.numpy as jnp
from jax import lax
from jax.experimental import pallas as pl
from jax.experimental.pallas import tpu as pltpu
from jax.experimental.pallas import tpu_sc as plsc

VOCAB = 1000000
EMBED_DIM = 64
BATCH = 16384
HIST = 50

NC = 2
NS = 16
NW = NC * NS

B_PER_W = BATCH // NW
BB = BATCH // 128
BB_W = B_PER_W // 128
CHUNK = 256
BB_C = CHUNK // 128
NCHUNK = 2 * HIST

_mesh = plsc.VectorSubcoreMesh(core_axis_name="c", subcore_axis_name="s")


@functools.partial(
    pl.kernel,
    mesh=_mesh,
    compiler_params=pltpu.CompilerParams(
        use_tc_tiling_on_sc=False, needs_layout_passes=False
    ),
    out_type=jax.ShapeDtypeStruct((HIST, 8, BB, 8, 128), jnp.float32),
    scratch_types=[
        pltpu.VMEM((HIST, B_PER_W), jnp.int32),
        pltpu.VMEM((CHUNK, 2 * EMBED_DIM), jnp.float32),
        pltpu.VMEM((CHUNK, 2 * EMBED_DIM), jnp.float32),
        pltpu.VMEM((BB_C, 8, 8, 129), jnp.float32),
        pltpu.VMEM((BB_C, 8, 8, 129), jnp.float32),
        pltpu.SemaphoreType.DMA,
        pltpu.SemaphoreType.DMA,
        pltpu.SemaphoreType.DMA,
        pltpu.SemaphoreType.DMA,
    ],
)
def _gather_kernel(
    tok_hbm, table_hbm, out_hbm,
    idx_v, rows0, rows1, tr0, tr1,
    gsem0, gsem1, wsem0, wsem1,
):
    wid = lax.axis_index("s") * NC + lax.axis_index("c")
    base = wid * B_PER_W
    gsems = (gsem0, gsem1)
    wsems = (wsem0, wsem1)
    rowbufs = (rows0, rows1)
    trbufs = (tr0, tr1)

    pltpu.sync_copy(tok_hbm.at[:, pl.ds(base, B_PER_W)], idx_v)

    lane = lax.iota(jnp.int32, 16)
    lane_hi = lane // 8
    lane_lo = lane % 8

    def _idx_slice(k):
        return idx_v.at[k // 2, pl.ds((k % 2) * CHUNK, CHUNK)]

    def _start_gather(k, b):
        pltpu.async_copy(table_hbm.at[_idx_slice(k)], rowbufs[b], gsems[b])

    def _wait_gather(k, b):
        pltpu.make_async_copy(
            table_hbm.at[_idx_slice(k)], rowbufs[b], gsems[b]
        ).wait()

    def _transpose(b):
        rows = rowbufs[b]
        tr = trbufs[b]

        @plsc.parallel_loop(0, CHUNK // 16, unroll=1)
        def vbody(v):
            bblv = jnp.full((16,), v // 8, jnp.int32)
            for b16 in range(16):
                b_row = v * 16 + b16
                biv = jnp.full((16,), (v % 8) * 16 + b16, jnp.int32)
                for c0 in range(0, EMBED_DIM, 16):
                    vals = rows[b_row, pl.ds(c0, 16)]
                    plsc.store_scatter(
                        tr, [bblv, lane_hi + (c0 // 8), lane_lo, biv], vals
                    )

    def _wb_copies(k, b):
        h = k // 2
        half = k % 2
        for bbl in range(BB_C):
            yield (
                trbufs[b].at[bbl, :, :, pl.ds(0, 128)],
                out_hbm.at[h, :, wid * BB_W + half * BB_C + bbl],
            )

    def _start_writeback(k, b):
        for src, dst in _wb_copies(k, b):
            pltpu.async_copy(src, dst, wsems[b])

    def _wait_writeback(k, b):
        for src, dst in _wb_copies(k, b):
            pltpu.make_async_copy(src, dst, wsems[b]).wait()

    for b in range(2):
        _start_gather(b, b)

    def _body(i, carry):
        for b in range(2):
            k = 2 * i + b
            _wait_gather(k, b)

            @pl.when(k >= 2)
            def _():
                _wait_writeback(k - 2, b)

            _transpose(b)
            _start_writeback(k, b)

            @pl.when(k + 2 < NCHUNK)
            def _():
                _start_gather(k + 2, b)

        return carry

    lax.fori_loop(0, NCHUNK // 2, _body, 0)

    for b in range(2):
        _wait_writeback(NCHUNK - 2 + b, b)


def kernel(token_id, embed_weight):
    tbl = jnp.pad(embed_weight, ((0, 0), (0, EMBED_DIM)))
    p5 = _gather_kernel(token_id.T, tbl)
    return p5.transpose(2, 4, 0, 1, 3).reshape(BATCH, HIST, EMBED_DIM)
